# baseline (speedup 1.0000x reference)
import functools

import jax
import jax.numpy as jnp
from jax import lax
from jax.experimental import pallas as pl
from jax.experimental.pallas import tpu as pltpu
from jax.experimental.pallas import tpu_sc as plsc

BATCH = 16384
EMBED = 64

_NC = 2
_NS = 16
_NW = _NC * _NS
_BPW = BATCH // _NW
_CHUNK = 128
_NCHUNK = _BPW // _CHUNK


def _sc_gather_body(uid_hbm, iid_hbm, ut_hbm, it_hbm, uout, iout,
                    uidx, iidx, urows, irows, sem):
    wid = lax.axis_index("s") * _NC + lax.axis_index("c")
    base = wid * _BPW
    pltpu.sync_copy(uid_hbm.at[pl.ds(base, _BPW)], uidx)
    pltpu.sync_copy(iid_hbm.at[pl.ds(base, _BPW)], iidx)
    copies = []
    for j in range(_NCHUNK):
        sl = pl.ds(j * _CHUNK, _CHUNK)
        copies.append(pltpu.async_copy(ut_hbm.at[uidx.at[sl]], urows.at[sl], sem))
        copies.append(pltpu.async_copy(it_hbm.at[iidx.at[sl]], irows.at[sl], sem))
    for c in copies:
        c.wait()
    pltpu.sync_copy(urows, uout.at[pl.ds(base, _BPW)])
    pltpu.sync_copy(irows, iout.at[pl.ds(base, _BPW)])


@jax.jit
def _sc_gather(user_ids, item_ids, user_table, item_table):
    mesh = plsc.VectorSubcoreMesh(core_axis_name="c", subcore_axis_name="s")
    k = functools.partial(
        pl.kernel,
        mesh=mesh,
        out_type=[
            jax.ShapeDtypeStruct((BATCH, EMBED), jnp.float32),
            jax.ShapeDtypeStruct((BATCH, EMBED), jnp.float32),
        ],
        scratch_types=[
            pltpu.VMEM((_BPW,), jnp.int32),
            pltpu.VMEM((_BPW,), jnp.int32),
            pltpu.VMEM((_BPW, EMBED), jnp.float32),
            pltpu.VMEM((_BPW, EMBED), jnp.float32),
            pltpu.SemaphoreType.DMA,
        ],
        compiler_params=pltpu.CompilerParams(use_tc_tiling_on_sc=False),
    )(_sc_gather_body)
    return k(user_ids, item_ids, user_table, item_table)


def _mlp_body(ue_ref, ie_ref, w1u_ref, w1i_ref, b1_ref, w2_ref, b2_ref, out_ref):
    h = (jnp.dot(ue_ref[...], w1u_ref[...], preferred_element_type=jnp.float32)
         + jnp.dot(ie_ref[...], w1i_ref[...], preferred_element_type=jnp.float32)
         + b1_ref[...])
    h = jax.nn.sigmoid(h)
    out_ref[...] = (jnp.sum(h * w2_ref[...], axis=1, keepdims=True)
                    + b2_ref[...])


_MLP_BLOCK = 2048


@jax.jit
def _tc_mlp(ue, ie, W1, b1, W2, b2):
    w1u = W1[:EMBED]
    w1i = W1[EMBED:]
    b1r = b1.reshape(1, EMBED)
    w2r = W2.reshape(1, EMBED)
    b2r = b2.reshape(1, 1)
    grid = (BATCH // _MLP_BLOCK,)
    return pl.pallas_call(
        _mlp_body,
        grid=grid,
        in_specs=[
            pl.BlockSpec((_MLP_BLOCK, EMBED), lambda i: (i, 0)),
            pl.BlockSpec((_MLP_BLOCK, EMBED), lambda i: (i, 0)),
            pl.BlockSpec((EMBED, EMBED), lambda i: (0, 0)),
            pl.BlockSpec((EMBED, EMBED), lambda i: (0, 0)),
            pl.BlockSpec((1, EMBED), lambda i: (0, 0)),
            pl.BlockSpec((1, EMBED), lambda i: (0, 0)),
            pl.BlockSpec((1, 1), lambda i: (0, 0)),
        ],
        out_specs=pl.BlockSpec((_MLP_BLOCK, 1), lambda i: (i, 0)),
        out_shape=jax.ShapeDtypeStruct((BATCH, 1), jnp.float32),
    )(ue, ie, w1u, w1i, b1r, w2r, b2r)


def kernel(user_ids, item_ids, user_table, item_table, W1, b1, W2, b2):
    ue, ie = _sc_gather(user_ids, item_ids, user_table, item_table)
    return _tc_mlp(ue, ie, W1, b1, W2, b2)


# speedup vs baseline: 1.5717x; 1.5717x over previous
import functools

import jax
import jax.numpy as jnp
from jax import lax
from jax.experimental import pallas as pl
from jax.experimental.pallas import tpu as pltpu
from jax.experimental.pallas import tpu_sc as plsc

BATCH = 16384
EMBED = 64

_NC = 2
_NS = 16
_NW = _NC * _NS
_BPW = BATCH // _NW
_BSZ = 16
_NB = _BPW // _BSZ


_HALF = _BPW // 2
_NBH = _HALF // _BSZ


def _sc_gather_body(uid_hbm, iid_hbm, ut_hbm, it_hbm, uout, iout,
                    uidx_s, iidx_s, urows, irows, sem):
    wid = lax.axis_index("s") * _NC + lax.axis_index("c")
    base = wid * _BPW
    pltpu.sync_copy(uid_hbm.at[pl.ds(base, _BPW)], uidx_s)
    pltpu.sync_copy(iid_hbm.at[pl.ds(base, _BPW)], iidx_s)

    def issue_batch(half_off, b):
        off = b * _BSZ
        uvec = uidx_s[pl.ds(half_off + off, _BSZ)]
        ivec = iidx_s[pl.ds(half_off + off, _BSZ)]
        for j in range(_BSZ):
            u = uvec[j]
            pltpu.async_copy(ut_hbm.at[pl.ds(u, 1)], urows.at[pl.ds(off + j, 1)], sem)
            v = ivec[j]
            pltpu.async_copy(it_hbm.at[pl.ds(v, 1)], irows.at[pl.ds(off + j, 1)], sem)

    def drain_batch():
        pltpu.make_async_copy(ut_hbm.at[pl.ds(0, 2 * _BSZ)], urows.at[pl.ds(0, 2 * _BSZ)], sem).wait()

    for half in range(2):
        half_off = half * _HALF

        @pl.loop(0, _NBH)
        def _loop(b):
            issue_batch(half_off, b)

            @pl.when(b >= 2)
            def _():
                drain_batch()

        drain_batch()
        drain_batch()
        pltpu.sync_copy(urows, uout.at[pl.ds(base + half_off, _HALF)])
        pltpu.sync_copy(irows, iout.at[pl.ds(base + half_off, _HALF)])


@jax.jit
def _sc_gather(user_ids, item_ids, user_table, item_table):
    mesh = plsc.VectorSubcoreMesh(core_axis_name="c", subcore_axis_name="s")
    k = functools.partial(
        pl.kernel,
        mesh=mesh,
        out_type=[
            jax.ShapeDtypeStruct((BATCH, EMBED), jnp.float32),
            jax.ShapeDtypeStruct((BATCH, EMBED), jnp.float32),
        ],
        scratch_types=[
            pltpu.VMEM((_BPW,), jnp.int32),
            pltpu.VMEM((_BPW,), jnp.int32),
            pltpu.VMEM((_HALF, EMBED), jnp.float32),
            pltpu.VMEM((_HALF, EMBED), jnp.float32),
            pltpu.SemaphoreType.DMA,
        ],
    )(_sc_gather_body)
    return k(user_ids, item_ids, user_table, item_table)


def _mlp_body(ue_ref, ie_ref, w1u_ref, w1i_ref, b1_ref, w2_ref, b2_ref, out_ref):
    h = (jnp.dot(ue_ref[...], w1u_ref[...], preferred_element_type=jnp.float32)
         + jnp.dot(ie_ref[...], w1i_ref[...], preferred_element_type=jnp.float32)
         + b1_ref[...])
    h = jax.nn.sigmoid(h)
    out_ref[...] = (jnp.sum(h * w2_ref[...], axis=1, keepdims=True)
                    + b2_ref[...])


_MLP_BLOCK = 2048


@jax.jit
def _tc_mlp(ue, ie, W1, b1, W2, b2):
    w1u = W1[:EMBED]
    w1i = W1[EMBED:]
    b1r = b1.reshape(1, EMBED)
    w2r = W2.reshape(1, EMBED)
    b2r = b2.reshape(1, 1)
    grid = (BATCH // _MLP_BLOCK,)
    return pl.pallas_call(
        _mlp_body,
        grid=grid,
        in_specs=[
            pl.BlockSpec((_MLP_BLOCK, EMBED), lambda i: (i, 0)),
            pl.BlockSpec((_MLP_BLOCK, EMBED), lambda i: (i, 0)),
            pl.BlockSpec((EMBED, EMBED), lambda i: (0, 0)),
            pl.BlockSpec((EMBED, EMBED), lambda i: (0, 0)),
            pl.BlockSpec((1, EMBED), lambda i: (0, 0)),
            pl.BlockSpec((1, EMBED), lambda i: (0, 0)),
            pl.BlockSpec((1, 1), lambda i: (0, 0)),
        ],
        out_specs=pl.BlockSpec((_MLP_BLOCK, 1), lambda i: (i, 0)),
        out_shape=jax.ShapeDtypeStruct((BATCH, 1), jnp.float32),
    )(ue, ie, w1u, w1i, b1r, w2r, b2r)


def kernel(user_ids, item_ids, user_table, item_table, W1, b1, W2, b2):
    ue, ie = _sc_gather(user_ids, item_ids, user_table, item_table)
    return _tc_mlp(ue, ie, W1, b1, W2, b2)
